# baseline (speedup 1.0000x reference)
import functools

import jax
import jax.numpy as jnp
from jax import lax
from jax.experimental import pallas as pl
from jax.experimental.pallas import tpu as pltpu
from jax.experimental.pallas import tpu_sc as plsc

N_ROWS = 100000
N_FACTORS = 64
BATCH = 16384

NUM_WORKERS = 32
B_PER_W = BATCH // NUM_WORKERS
CHUNK = 128
N_CHUNKS = B_PER_W // CHUNK
LANES = 16
PACK = 2 * N_FACTORS
HALF_CHUNKS = N_CHUNKS // 2
GROUPS_PER_HALF = (B_PER_W // 2) // LANES

UNITS = N_ROWS // 16
BASE_UNITS = UNITS // NUM_WORKERS
EXTRA_TILES = UNITS % NUM_WORKERS
SLAB_UNITS = 8
FULL_SLABS = BASE_UNITS // SLAB_UNITS
SLAB_ROWS = SLAB_UNITS * 16


def _make_relayout_kernel():
    mesh = plsc.VectorSubcoreMesh(core_axis_name="c", subcore_axis_name="s")

    @functools.partial(
        pl.kernel,
        mesh=mesh,
        compiler_params=pltpu.CompilerParams(needs_layout_passes=False),
        out_type=[
            jax.ShapeDtypeStruct((N_ROWS // 2, PACK), jnp.float32),
            jax.ShapeDtypeStruct((N_ROWS // 2, PACK), jnp.float32),
        ],
        scratch_types=[
            pltpu.VMEM((SLAB_ROWS, N_FACTORS), jnp.float32),
            pltpu.VMEM((SLAB_ROWS // 2, PACK), jnp.float32),
        ],
    )
    def relayout(d_tab, g_tab, dp_out, gp_out, va, vb):
        wid = lax.axis_index("s") * 2 + lax.axis_index("c")
        u0 = wid * BASE_UNITS + jnp.minimum(wid, EXTRA_TILES)
        nu = BASE_UNITS + jnp.where(wid < EXTRA_TILES, 1, 0)

        def compact_slab():
            for r in range(SLAB_ROWS):
                for k in range(N_FACTORS // LANES):
                    pos = r * N_FACTORS + k * LANES
                    vb[pos // PACK, pl.ds(pos % PACK, LANES)] = (
                        va[r, pl.ds(k * LANES, LANES)])

        def do_table(tab, out):
            def slab(s, carry):
                row0 = pl.multiple_of((u0 + s * SLAB_UNITS) * 16, 16)
                pltpu.sync_copy(tab.at[pl.ds(row0, SLAB_ROWS)], va)
                compact_slab()
                pltpu.sync_copy(vb, out.at[pl.ds(pl.multiple_of(row0 // 2, 8),
                                                 SLAB_ROWS // 2)])
                return carry

            lax.fori_loop(0, FULL_SLABS, slab, 0)

            def rem_unit(r_u, carry):
                row0 = pl.multiple_of(
                    (u0 + FULL_SLABS * SLAB_UNITS + r_u) * 16, 16)
                pltpu.sync_copy(tab.at[pl.ds(row0, 16)],
                                va.at[pl.ds(0, 16)])
                for r in range(16):
                    for k in range(N_FACTORS // LANES):
                        pos = r * N_FACTORS + k * LANES
                        vb[pos // PACK, pl.ds(pos % PACK, LANES)] = (
                            va[r, pl.ds(k * LANES, LANES)])
                pltpu.sync_copy(vb.at[pl.ds(0, 8)],
                                out.at[pl.ds(pl.multiple_of(row0 // 2, 8), 8)])
                return carry

            lax.fori_loop(0, nu - FULL_SLABS * SLAB_UNITS, rem_unit, 0)

        do_table(d_tab, dp_out)
        do_table(g_tab, gp_out)

    return relayout


def _make_gather_kernel():
    mesh = plsc.VectorSubcoreMesh(core_axis_name="c", subcore_axis_name="s")

    @functools.partial(
        pl.kernel,
        mesh=mesh,
        compiler_params=pltpu.CompilerParams(needs_layout_passes=False),
        out_type=jax.ShapeDtypeStruct((BATCH,), jnp.float32),
        scratch_types=[
            pltpu.VMEM((N_CHUNKS, CHUNK), jnp.int32),
            pltpu.VMEM((N_CHUNKS, CHUNK), jnp.int32),
            pltpu.VMEM((N_CHUNKS, CHUNK), jnp.int32),
            pltpu.VMEM((N_CHUNKS, CHUNK), jnp.int32),
            pltpu.VMEM((B_PER_W // 2, PACK), jnp.float32),
            pltpu.VMEM((B_PER_W // 2, PACK), jnp.float32),
            pltpu.VMEM((2 * N_FACTORS,), jnp.float32),
            pltpu.VMEM((LANES,), jnp.float32),
            pltpu.VMEM((B_PER_W,), jnp.float32),
            pltpu.SemaphoreType.DMA,
        ],
    )
    def sc_body(dis_tab, gene_tab, dis_idx, gene_idx, w_hbm, b_hbm, out_hbm,
                idx_d, idx_g, par_d, par_g, drows, grows, wv, bv, outv, sem):
        wid = lax.axis_index("s") * 2 + lax.axis_index("c")
        base = wid * B_PER_W
        pltpu.sync_copy(w_hbm, wv)
        pltpu.sync_copy(b_hbm, bv)
        for c in range(N_CHUNKS):
            pltpu.sync_copy(dis_idx.at[pl.ds(base + c * CHUNK, CHUNK)],
                            idx_d.at[c])
            pltpu.sync_copy(gene_idx.at[pl.ds(base + c * CHUNK, CHUNK)],
                            idx_g.at[c])
        for c in range(N_CHUNKS):
            for l in range(CHUNK // LANES):
                sl = pl.ds(l * LANES, LANES)
                vd = idx_d[c, sl]
                vg = idx_g[c, sl]
                par_d[c, sl] = lax.bitwise_and(vd, 1)
                par_g[c, sl] = lax.bitwise_and(vg, 1)
                idx_d[c, sl] = lax.shift_right_logical(vd, 1)
                idx_g[c, sl] = lax.shift_right_logical(vg, 1)

        wd = [wv[pl.ds(k * LANES, LANES)] for k in range(4)]
        wg = [wv[pl.ds(N_FACTORS + k * LANES, LANES)] for k in range(4)]
        bvec = bv[...]
        lane = lax.iota(jnp.int32, LANES)

        for half in range(2):
            copies = []
            for cc in range(HALF_CHUNKS):
                c = half * HALF_CHUNKS + cc
                copies.append(pltpu.async_copy(
                    dis_tab.at[idx_d.at[c]],
                    drows.at[pl.ds(cc * CHUNK, CHUNK)], sem))
                copies.append(pltpu.async_copy(
                    gene_tab.at[idx_g.at[c]],
                    grows.at[pl.ds(cc * CHUNK, CHUNK)], sem))
            for cp in copies:
                cp.wait()

            def group(g, carry, half=half):
                nsub = CHUNK // LANES
                c = half * HALF_CHUNKS + g // nsub
                sl = pl.ds((g % nsub) * LANES, LANES)
                pv_d = par_d[c, sl] * N_FACTORS
                pv_g = par_g[c, sl] * N_FACTORS
                acc = bvec
                for r in range(LANES):
                    row = g * LANES + r
                    od = pv_d[r]
                    og = pv_g[r]
                    p = drows[row, pl.ds(od, LANES)] * wd[0]
                    p = p + grows[row, pl.ds(og, LANES)] * wg[0]
                    for k in range(1, 4):
                        p = p + drows[row, pl.ds(od + k * LANES, LANES)] * wd[k]
                        p = p + grows[row, pl.ds(og + k * LANES, LANES)] * wg[k]
                    s = jnp.sum(p)
                    acc = jnp.where(lane == r, acc + s, acc)
                outv[pl.ds(half * (B_PER_W // 2) + g * LANES, LANES)] = (
                    1.0 / (1.0 + jnp.exp(-acc)))
                return carry

            lax.fori_loop(0, GROUPS_PER_HALF, group, 0)

        pltpu.sync_copy(outv, out_hbm.at[pl.ds(base, B_PER_W)])

    return sc_body


_relayout = _make_relayout_kernel()
_sc_gather = _make_gather_kernel()


def kernel(diseases, genes, disease_table, gene_table, W, b):
    dp, gp = _relayout(disease_table, gene_table)
    w_flat = W.reshape(2 * N_FACTORS)
    b_vec = jnp.broadcast_to(b, (LANES,))
    return _sc_gather(dp, gp, diseases, genes, w_flat, b_vec)


# speedup vs baseline: 1.5577x; 1.5577x over previous
import functools

import jax
import jax.numpy as jnp
from jax import lax
from jax.experimental import pallas as pl
from jax.experimental.pallas import tpu as pltpu
from jax.experimental.pallas import tpu_sc as plsc

N_ROWS = 100000
N_FACTORS = 64
BATCH = 16384

NUM_WORKERS = 32
B_PER_W = BATCH // NUM_WORKERS
CHUNK = 128
N_CHUNKS = B_PER_W // CHUNK
LANES = 16
N_GROUPS = B_PER_W // LANES
KSUB = N_FACTORS // LANES

_MESH = plsc.VectorSubcoreMesh(core_axis_name="c", subcore_axis_name="s")
_PARAMS = pltpu.CompilerParams(needs_layout_passes=False,
                               use_tc_tiling_on_sc=False)
_SCRATCH = [
    pltpu.VMEM((N_CHUNKS, CHUNK), jnp.int32),
    pltpu.VMEM((B_PER_W, N_FACTORS), jnp.float32),
    pltpu.VMEM((N_FACTORS,), jnp.float32),
    pltpu.VMEM((B_PER_W,), jnp.float32),
    pltpu.VMEM((B_PER_W,), jnp.float32),
    pltpu.SemaphoreType.DMA,
]


def _stage(tab, t_idx, w_hbm, idx, rows, wv, base, sem):
    pltpu.sync_copy(w_hbm, wv)
    for c in range(N_CHUNKS):
        pltpu.sync_copy(t_idx.at[pl.ds(base + c * CHUNK, CHUNK)], idx.at[c])
    copies = []
    for c in range(N_CHUNKS):
        copies.append(pltpu.async_copy(
            tab.at[idx.at[c]], rows.at[pl.ds(c * CHUNK, CHUNK)], sem))
    for cp in copies:
        cp.wait()


def _dots(rows, wv, carry_vec, outv, finalize):
    w = [wv[pl.ds(k * LANES, LANES)] for k in range(KSUB)]
    lane = lax.iota(jnp.int32, LANES)

    def group(g, c):
        acc = carry_vec[pl.ds(g * LANES, LANES)]
        for r in range(LANES):
            row = g * LANES + r
            p = rows[row, pl.ds(0, LANES)] * w[0]
            for k in range(1, KSUB):
                p = p + rows[row, pl.ds(k * LANES, LANES)] * w[k]
            s = jnp.sum(p)
            acc = jnp.where(lane == r, acc + s, acc)
        if finalize:
            acc = 1.0 / (1.0 + jnp.exp(-acc))
        outv[pl.ds(g * LANES, LANES)] = acc
        return c

    lax.fori_loop(0, N_GROUPS, group, 0)


@functools.partial(
    pl.kernel, mesh=_MESH, compiler_params=_PARAMS,
    out_type=jax.ShapeDtypeStruct((BATCH,), jnp.float32),
    scratch_types=_SCRATCH)
def _disease_partial(dis_tab, dis_idx, w_hbm, b_hbm, part_hbm,
                     idx, rows, wv, cin, outv, sem):
    wid = lax.axis_index("s") * 2 + lax.axis_index("c")
    base = wid * B_PER_W
    _stage(dis_tab, dis_idx, w_hbm, idx, rows, wv, base, sem)
    pltpu.sync_copy(b_hbm.at[pl.ds(base, B_PER_W)], cin)
    _dots(rows, wv, cin, outv, finalize=False)
    pltpu.sync_copy(outv, part_hbm.at[pl.ds(base, B_PER_W)])


@functools.partial(
    pl.kernel, mesh=_MESH, compiler_params=_PARAMS,
    out_type=jax.ShapeDtypeStruct((BATCH,), jnp.float32),
    scratch_types=_SCRATCH)
def _gene_final(gene_tab, gene_idx, w_hbm, part_hbm, out_hbm,
                idx, rows, wv, cin, outv, sem):
    wid = lax.axis_index("s") * 2 + lax.axis_index("c")
    base = wid * B_PER_W
    _stage(gene_tab, gene_idx, w_hbm, idx, rows, wv, base, sem)
    pltpu.sync_copy(part_hbm.at[pl.ds(base, B_PER_W)], cin)
    _dots(rows, wv, cin, outv, finalize=True)
    pltpu.sync_copy(outv, out_hbm.at[pl.ds(base, B_PER_W)])


def kernel(diseases, genes, disease_table, gene_table, W, b):
    wd = W[0, :N_FACTORS]
    wg = W[0, N_FACTORS:]
    b_vec = jnp.broadcast_to(b, (BATCH,))
    part = _disease_partial(disease_table, diseases, wd, b_vec)
    return _gene_final(gene_table, genes, wg, part)
